# baseline (speedup 1.0000x reference)
import functools

import jax
import jax.numpy as jnp
from jax import lax
from jax.experimental import pallas as pl
from jax.experimental.pallas import tpu as pltpu
from jax.experimental.pallas import tpu_sc as plsc

NUM_EMB = 1000000
DIM = 32

BATCH = 4096
SEQ = 200
LANE = 16
BBLK = 128
NT = 5
NBUF = 2


def _make_kernel():
    info = plsc.get_sparse_core_info()
    NC, NS = info.num_cores, info.num_subcores
    NW = NC * NS
    assert BATCH // BBLK == NW
    chunks = SEQ // NT
    outer = chunks // NBUF

    mesh = plsc.VectorSubcoreMesh(core_axis_name="c", subcore_axis_name="s")

    @functools.partial(
        pl.kernel,
        mesh=mesh,
        out_type=jax.ShapeDtypeStruct(
            (SEQ, DIM // 8, BATCH // BBLK, 8, BBLK), jnp.float32
        ),
        scratch_types=[
            pltpu.VMEM((3, NT, BBLK), jnp.int32),
            pltpu.VMEM((NBUF, NT, BBLK, DIM), jnp.float32),
            pltpu.VMEM((NBUF, NT, DIM // 8, 8, BBLK), jnp.float32),
            pltpu.SemaphoreType.DMA((NBUF,)),
            pltpu.SemaphoreType.DMA((NBUF,)),
            pltpu.SemaphoreType.DMA,
        ],
        compiler_params=pltpu.CompilerParams(
            use_tc_tiling_on_sc=False, needs_layout_passes=False
        ),
    )
    def emb(idx_hbm, table_hbm, out_hbm, idx_v, rows_v, tbuf, gsem, osem, isem):
        wid = lax.axis_index("s") * NC + lax.axis_index("c")
        b0 = pl.multiple_of(wid * BBLK, BBLK)

        def idx_slab(c):
            t0 = pl.multiple_of(c * NT, NT)
            return idx_hbm.at[pl.ds(t0, NT), pl.ds(b0, BBLK)]

        def fetch_idx(c):
            pltpu.async_copy(idx_slab(c), idx_v.at[lax.rem(c, 3)], isem)

        def fire_gather(c, db):
            pltpu.make_async_copy(idx_slab(c), idx_v.at[lax.rem(c, 3)], isem).wait()
            ib = lax.rem(c, 3)
            for jt in range(NT):
                pltpu.async_copy(
                    table_hbm.at[idx_v.at[ib].at[jt]],
                    rows_v.at[db].at[jt],
                    gsem.at[db],
                )

        def wait_gather_jt(db, jt):
            pltpu.make_async_copy(
                table_hbm.at[idx_v.at[0].at[jt]],
                rows_v.at[db].at[jt],
                gsem.at[db],
            ).wait()

        TRUE16 = jnp.full((LANE,), True)

        def transpose_chunk(c, db):
            t0 = pl.multiple_of(c * NT, NT)
            lanes = lax.iota(jnp.int32, LANE)
            for jt in range(NT):
                wait_gather_jt(db, jt)
                rows = rows_v.at[db].at[jt]
                tb = tbuf.at[db].at[jt]

                @plsc.parallel_loop(0, DIM, unroll=2)
                def _(f):
                    fvec = jnp.full((LANE,), f, jnp.int32)
                    trow = tb.at[f // 8].at[f % 8]
                    for jb in range(BBLK // LANE):
                        bvec = jb * LANE + lanes
                        v = plsc.load_gather(rows, [bvec, fvec])
                        plsc.store_compressed(
                            trow.at[pl.ds(jb * LANE, LANE)], v, mask=TRUE16
                        )

                pltpu.async_copy(
                    tb,
                    out_hbm.at[t0 + jt, pl.ds(0, DIM // 8), wid],
                    osem.at[db],
                )

        def wait_out(db):
            for _ in range(NT):
                pltpu.make_async_copy(
                    tbuf.at[db].at[0],
                    out_hbm.at[0, pl.ds(0, DIM // 8), 0],
                    osem.at[db],
                ).wait()

        fetch_idx(0)
        fire_gather(0, 0)
        fetch_idx(1)

        def body(g0, carry):
            for db in range(NBUF):
                g = g0 * NBUF + db
                dn = (db + 1) % NBUF

                @pl.when(g >= 1)
                def _():
                    wait_out(dn)

                @pl.when(g + 2 < chunks)
                def _():
                    fetch_idx(g + 2)

                @pl.when(g + 1 < chunks)
                def _():
                    fire_gather(g + 1, dn)

                transpose_chunk(g, db)
            return carry

        lax.fori_loop(0, outer, body, 0)
        wait_out((chunks - 1) % NBUF)

    return emb


_emb_kernel = _make_kernel()


@jax.jit
def kernel(inputs, weight):
    idx_t = inputs.T.astype(jnp.int32)
    out5 = _emb_kernel(idx_t, weight)
    return out5.transpose(2, 4, 0, 1, 3).reshape(BATCH, SEQ, DIM)


# speedup vs baseline: 1.0055x; 1.0055x over previous
import functools

import jax
import jax.numpy as jnp
from jax import lax
from jax.experimental import pallas as pl
from jax.experimental.pallas import tpu as pltpu
from jax.experimental.pallas import tpu_sc as plsc

NUM_EMB = 1000000
DIM = 32

BATCH = 4096
SEQ = 200
LANE = 16
BBLK = 128
NT = 5
NBUF = 2


def _make_kernel():
    info = plsc.get_sparse_core_info()
    NC, NS = info.num_cores, info.num_subcores
    NW = NC * NS
    assert BATCH // BBLK == NW
    chunks = SEQ // NT
    outer = chunks // NBUF

    mesh = plsc.VectorSubcoreMesh(core_axis_name="c", subcore_axis_name="s")

    @functools.partial(
        pl.kernel,
        mesh=mesh,
        out_type=jax.ShapeDtypeStruct(
            (SEQ, DIM // 8, BATCH // BBLK, 8, BBLK), jnp.float32
        ),
        scratch_types=[
            pltpu.VMEM((3, NT, BBLK), jnp.int32),
            pltpu.VMEM((NBUF, NT, BBLK, DIM), jnp.float32),
            pltpu.VMEM((NBUF, NT, DIM // 8, 8, BBLK), jnp.float32),
            pltpu.SemaphoreType.DMA((NBUF,)),
            pltpu.SemaphoreType.DMA((NBUF,)),
            pltpu.SemaphoreType.DMA,
        ],
        compiler_params=pltpu.CompilerParams(
            use_tc_tiling_on_sc=False, needs_layout_passes=False
        ),
    )
    def emb(idx_hbm, table_hbm, out_hbm, idx_v, rows_v, tbuf, gsem, osem, isem):
        wid = lax.axis_index("s") * NC + lax.axis_index("c")
        b0 = pl.multiple_of(wid * BBLK, BBLK)

        def idx_slab(c):
            t0 = pl.multiple_of(c * NT, NT)
            return idx_hbm.at[pl.ds(t0, NT), pl.ds(b0, BBLK)]

        def fetch_idx(c):
            pltpu.async_copy(idx_slab(c), idx_v.at[lax.rem(c, 3)], isem)

        def fire_gather(c, db):
            pltpu.make_async_copy(idx_slab(c), idx_v.at[lax.rem(c, 3)], isem).wait()
            ib = lax.rem(c, 3)
            for jt in range(NT):
                pltpu.async_copy(
                    table_hbm.at[idx_v.at[ib].at[jt]],
                    rows_v.at[db].at[jt],
                    gsem.at[db],
                )

        def wait_gather_jt(db, jt):
            pltpu.make_async_copy(
                table_hbm.at[idx_v.at[0].at[jt]],
                rows_v.at[db].at[jt],
                gsem.at[db],
            ).wait()

        TRUE16 = jnp.full((LANE,), True)

        def transpose_chunk(c, db):
            t0 = pl.multiple_of(c * NT, NT)
            lanes = lax.iota(jnp.int32, LANE)
            for jt in range(NT):
                wait_gather_jt(db, jt)
                rows = rows_v.at[db].at[jt]
                tb = tbuf.at[db].at[jt]

                @plsc.parallel_loop(0, DIM, unroll=1)
                def _(f):
                    fvec = jnp.full((LANE,), f, jnp.int32)
                    trow = tb.at[f // 8].at[f % 8]
                    for jb in range(BBLK // LANE):
                        bvec = jb * LANE + lanes
                        v = plsc.load_gather(rows, [bvec, fvec])
                        plsc.store_compressed(
                            trow.at[pl.ds(jb * LANE, LANE)], v, mask=TRUE16
                        )

                pltpu.async_copy(
                    tb,
                    out_hbm.at[t0 + jt, pl.ds(0, DIM // 8), wid],
                    osem.at[db],
                )

        def wait_out(db):
            for _ in range(NT):
                pltpu.make_async_copy(
                    tbuf.at[db].at[0],
                    out_hbm.at[0, pl.ds(0, DIM // 8), 0],
                    osem.at[db],
                ).wait()

        fetch_idx(0)
        fire_gather(0, 0)
        fetch_idx(1)

        def body(g0, carry):
            for db in range(NBUF):
                g = g0 * NBUF + db
                dn = (db + 1) % NBUF

                @pl.when(g >= 1)
                def _():
                    wait_out(dn)

                @pl.when(g + 2 < chunks)
                def _():
                    fetch_idx(g + 2)

                @pl.when(g + 1 < chunks)
                def _():
                    fire_gather(g + 1, dn)

                transpose_chunk(g, db)
            return carry

        lax.fori_loop(0, outer, body, 0)
        wait_out((chunks - 1) % NBUF)

    return emb


_emb_kernel = _make_kernel()


@jax.jit
def kernel(inputs, weight):
    idx_t = inputs.T.astype(jnp.int32)
    out5 = _emb_kernel(idx_t, weight)
    return out5.transpose(2, 4, 0, 1, 3).reshape(BATCH, SEQ, DIM)
